# Initial kernel scaffold; baseline (speedup 1.0000x reference)
import jax
import jax.numpy as jnp
from jax.experimental import pallas as pl


def kernel(x, edge_index, batch, edge_attr, Wl1, Wr1, We1, att1, b1, Wl2, Wr2, We2, att2, b2, lin1_W, lin1_b, bn_g, bn_b, lin2_W, lin2_b):
    raise NotImplementedError("write your pallas kernel here")


# speedup vs baseline: 4.6004x; 4.6004x over previous
import functools

import jax
import jax.numpy as jnp
from jax import lax
from jax.experimental import pallas as pl
from jax.experimental.pallas import tpu as pltpu
from jax.experimental.pallas import tpu_sc as plsc

N = 10000
E = 320000
IN = 128
HID = 64
OUT = 10
ED = 16
G = 64

NC, NS, LANES = 2, 16, 16
NW = NC * NS
K = 64
NPAIR = -(-E // (NW * 2 * K))
NCHUNK = 2 * NPAIR
EPW = NCHUNK * K
E_PAD = EPW * NW
NPAD = 10240
RPT = NPAD // NS
DENW = 16
HID2 = 2 * HID
BLK = 1000
NBLK = N // BLK


def _bf16_round(x):
    bits = plsc.bitcast(x, jnp.int32)
    one = jnp.full(bits.shape, 1, jnp.int32)
    sixteen = jnp.full(bits.shape, 16, jnp.int32)
    lsb = lax.shift_right_logical(bits, sixteen) & one
    rounded = bits + 0x7FFF + lsb
    return plsc.bitcast(rounded & jnp.int32(-65536), jnp.float32)


def _exp_f32(x):
    log2e = 1.4426950408889634
    half = jnp.where(x >= 0, 0.5, -0.5)
    ni = (x * log2e + half).astype(jnp.int32)
    nf = ni.astype(jnp.float32)
    r = x - nf * 0.693359375 - nf * (-2.12194440e-4)
    p = 1.0 / 720.0
    p = p * r + 1.0 / 120.0
    p = p * r + 1.0 / 24.0
    p = p * r + 1.0 / 6.0
    p = p * r + 0.5
    p = p * r + 1.0
    p = p * r + 1.0
    scale = plsc.bitcast(
        lax.shift_left(ni + 127, jnp.full(ni.shape, 23, jnp.int32)),
        jnp.float32)
    return p * scale


@functools.cache
def _get_sc_edge_pass():
  mesh = plsc.VectorSubcoreMesh(core_axis_name="c", subcore_axis_name="s",
                                num_cores=NC, num_subcores=NS)

  @functools.partial(
    pl.kernel,
    out_type=jax.ShapeDtypeStruct((NC, NPAD, HID2), jnp.float32),
    mesh=mesh,
    scratch_types=[
        pltpu.VMEM((2 * K,), jnp.int32),
        pltpu.VMEM((K,), jnp.int32),
        pltpu.VMEM((K, HID2), jnp.float32),
        pltpu.VMEM((K, HID2), jnp.float32),
        pltpu.VMEM((HID, 2 * K), jnp.float32),
        pltpu.VMEM((K, HID2), jnp.float32),
        pltpu.VMEM((K,), jnp.float32),
        pltpu.VMEM((HID,), jnp.float32),
        pltpu.VMEM_SHARED((NPAD, HID2), jnp.float32),
        pltpu.SemaphoreType.DMA,
        pltpu.SemaphoreType.DMA,
    ],
    compiler_params=pltpu.CompilerParams(needs_layout_passes=False),
  )
  def _sc_edge_pass(sd_hbm, xlr_hbm, ewt_hbm, att_hbm, zn_hbm, acc_out,
                    sd_v, dsti_v, g1_v, g2_v, ewt_v, wr_v, lbuf, att_v,
                    sh_acc, sem1, sem2):
      c = lax.axis_index("c")
      s = lax.axis_index("s")
      wid = c * NS + s
      row0 = s * RPT
      zero16 = jnp.zeros((LANES,), jnp.float32)

      pltpu.sync_copy(zn_hbm, sh_acc.at[pl.ds(row0, RPT)])
      pltpu.sync_copy(att_hbm, att_v)

      def zrow(e, carry):
          for q in range(HID2 // LANES):
              wr_v[e, pl.ds(q * LANES, LANES)] = zero16
          return carry

      lax.fori_loop(0, K, zrow, 0)
      plsc.subcore_barrier()

      def pair(t, carry):
          pbase = (wid * NPAIR + t) * 2 * K
          pltpu.sync_copy(ewt_hbm.at[:, pl.ds(pbase, 2 * K)], ewt_v)
          for u in range(2):
              ci = (wid * NPAIR + t) * 2 + u
              base = ci * K
              pltpu.sync_copy(sd_hbm.at[ci], sd_v)
              for q in range(K // LANES):
                  dsti_v[pl.ds(q * LANES, LANES)] = (
                      sd_v[pl.ds(K + q * LANES, LANES)])
              cp1 = pltpu.async_copy(xlr_hbm.at[sd_v.at[pl.ds(0, K)]],
                                     g1_v, sem1)
              cp2 = pltpu.async_copy(xlr_hbm.at[dsti_v], g2_v, sem2)
              cp1.wait()
              cp2.wait()

              iota16 = lax.iota(jnp.int32, LANES)
              col64 = jnp.full((LANES,), HID, jnp.int32)
              att_regs = [_bf16_round(att_v[pl.ds(q * LANES, LANES)])
                          for q in range(HID // LANES)]

              def lgroup(g, carry2):
                  eids = g * LANES + iota16
                  acc = jnp.zeros((LANES,), jnp.float32)
                  for h in range(HID):
                      hv = jnp.full((LANES,), h, jnp.int32)
                      hv2 = jnp.full((LANES,), HID + h, jnp.int32)
                      m = (plsc.load_gather(g1_v, [eids, hv])
                           + plsc.load_gather(g2_v, [eids, hv2])
                           + ewt_v[h, pl.ds(u * K + g * LANES, LANES)])
                      m = jnp.maximum(m, 0.2 * m)
                      m = _bf16_round(m)
                      acc = acc + m * att_regs[h // LANES][h % LANES]
                  gi = base + g * LANES + iota16
                  ex = jnp.where(gi < E, _exp_f32(acc), 0.0)
                  lbuf[pl.ds(g * LANES, LANES)] = ex
                  plsc.store_scatter(wr_v, [eids, col64], ex)
                  return carry2

              lax.fori_loop(0, K // LANES, lgroup, 0)

              def wgroup(g, carry2):
                  exv = lbuf[pl.ds(g * LANES, LANES)]
                  for uu in range(LANES):
                      e = g * LANES + uu
                      w = exv[uu]
                      for q in range(HID // LANES):
                          sl = pl.ds(q * LANES, LANES)
                          wr_v[e, sl] = g1_v[e, sl] * w
                  return carry2

              lax.fori_loop(0, K // LANES, wgroup, 0)

              pltpu.sync_copy(wr_v, sh_acc.at[dsti_v], add=True)
          return carry

      lax.fori_loop(0, NPAIR, pair, 0)
      plsc.subcore_barrier()

      sl = pl.ds(row0, RPT)
      pltpu.sync_copy(sh_acc.at[sl], acc_out.at[c, sl])

  return _sc_edge_pass


def _mm_lr(x, wl, wr):
    n, d_in = x.shape

    def kern(x_ref, wl_ref, wr_ref, o_ref):
        xv = x_ref[...]
        o_ref[:, 0:HID] = jnp.dot(xv, wl_ref[...],
                                  preferred_element_type=jnp.float32,
                    precision=None)
        o_ref[:, HID:HID2] = jnp.dot(xv, wr_ref[...],
                                     preferred_element_type=jnp.float32,
                    precision=None)

    return pl.pallas_call(
        kern,
        grid=(n // BLK,),
        in_specs=[
            pl.BlockSpec((BLK, d_in), lambda i: (i, 0)),
            pl.BlockSpec((d_in, HID), lambda i: (0, 0)),
            pl.BlockSpec((d_in, HID), lambda i: (0, 0)),
        ],
        out_specs=pl.BlockSpec((BLK, HID2), lambda i: (i, 0)),
        out_shape=jax.ShapeDtypeStruct((n, HID2), jnp.float32),
    )(x, wl, wr)


def _edge_mm(ea_pad, eat_pad, we, wet):
    eblk = 4096
    ngrid = E_PAD // eblk

    def kern(ea_ref, eat_ref, we_ref, wet_ref, ewt_ref, cs_ref, csea):
        i = pl.program_id(0)
        ewt_ref[...] = jnp.dot(wet_ref[...], eat_ref[...],
                               preferred_element_type=jnp.float32,
                    precision=None)

        @pl.when(i == 0)
        def _():
            csea[...] = jnp.zeros_like(csea)

        csea[...] += jnp.sum(ea_ref[...], axis=0, keepdims=True)

        @pl.when(i == ngrid - 1)
        def _():
            cs_ref[...] = jnp.dot(csea[...], we_ref[...],
                                  preferred_element_type=jnp.float32,
                    precision=None)

    return pl.pallas_call(
        kern,
        grid=(ngrid,),
        in_specs=[
            pl.BlockSpec((eblk, ED), lambda i: (i, 0)),
            pl.BlockSpec((ED, eblk), lambda i: (0, i)),
            pl.BlockSpec((ED, HID), lambda i: (0, 0)),
            pl.BlockSpec((HID, ED), lambda i: (0, 0)),
        ],
        out_specs=[
            pl.BlockSpec((HID, eblk), lambda i: (0, i)),
            pl.BlockSpec((8, HID), lambda i: (0, 0)),
        ],
        out_shape=[
            jax.ShapeDtypeStruct((HID, E_PAD), jnp.float32),
            jax.ShapeDtypeStruct((8, HID), jnp.float32),
        ],
        scratch_shapes=[pltpu.VMEM((8, ED), jnp.float32)],
    )(ea_pad, eat_pad, we, wet)


def _self_and_norm(xlr, cs, att8, b8, acc_, relu):
    def kern(xlr_ref, cs_ref, att_ref, b_ref, acc_ref, h_ref):
        xlrv = xlr_ref[...]
        xlv = xlrv[:, 0:HID]
        xrv = xlrv[:, HID:HID2]
        mean_w = cs_ref[0:1, :] * (1.0 / E)
        ms = xlv + xrv + mean_w
        ms = jnp.maximum(ms, 0.2 * ms)
        ms = ms.astype(jnp.bfloat16).astype(jnp.float32)
        attv = att_ref[0:1, :].astype(jnp.bfloat16).astype(jnp.float32)
        sv = jnp.exp(jnp.sum(ms * attv, axis=1, keepdims=True))
        num = sv * xlv + acc_ref[0, :, 0:HID] + acc_ref[1, :, 0:HID]
        den = sv + acc_ref[0, :, HID:HID + 1] + acc_ref[1, :, HID:HID + 1]
        h = num / den + b_ref[0:1, :]
        if relu:
            h = jnp.maximum(h, 0.0)
        h_ref[...] = h

    return pl.pallas_call(
        functools.partial(kern),
        grid=(NBLK,),
        in_specs=[
            pl.BlockSpec((BLK, HID2), lambda i: (i, 0)),
            pl.BlockSpec((8, HID), lambda i: (0, 0)),
            pl.BlockSpec((8, HID), lambda i: (0, 0)),
            pl.BlockSpec((8, HID), lambda i: (0, 0)),
            pl.BlockSpec((NC, BLK, HID2), lambda i: (0, i, 0)),
        ],
        out_specs=pl.BlockSpec((BLK, HID), lambda i: (i, 0)),
        out_shape=jax.ShapeDtypeStruct((N, HID), jnp.float32),
    )(xlr, cs, att8, b8, acc_)


def _head(xlr2, cs2, att8, b8, acc2, batch3, l1w, l1b8, bng8, bnb8,
          l2wp, l2bp):
    def kern(xlr_ref, cs_ref, att_ref, b_ref, acc_ref, bat_ref,
             l1w_ref, l1b_ref, bng_ref, bnb_ref, l2w_ref, l2b_ref, out_ref,
             acc, cnt):
        i = pl.program_id(0)

        @pl.when(i == 0)
        def _():
            acc[...] = jnp.zeros_like(acc)
            cnt[...] = jnp.zeros_like(cnt)

        xlrv = xlr_ref[...]
        xlv = xlrv[:, 0:HID]
        xrv = xlrv[:, HID:HID2]
        mean_w = cs_ref[0:1, :] * (1.0 / E)
        ms = xlv + xrv + mean_w
        ms = jnp.maximum(ms, 0.2 * ms)
        ms = ms.astype(jnp.bfloat16).astype(jnp.float32)
        attv = att_ref[0:1, :].astype(jnp.bfloat16).astype(jnp.float32)
        sv = jnp.exp(jnp.sum(ms * attv, axis=1, keepdims=True))
        num = sv * xlv + acc_ref[0, :, 0:HID] + acc_ref[1, :, 0:HID]
        den = sv + acc_ref[0, :, HID:HID + 1] + acc_ref[1, :, HID:HID + 1]
        h = num / den + b_ref[0:1, :]

        one_hot = (bat_ref[0, 0:1, :] ==
                   lax.broadcasted_iota(jnp.int32, (G, BLK), 0)
                   ).astype(jnp.float32)
        acc[...] += jnp.dot(one_hot, h, preferred_element_type=jnp.float32,
                    precision=lax.Precision.HIGHEST)
        cnt[:, 0:1] += jnp.sum(one_hot, axis=1, keepdims=True)

        @pl.when(i == NBLK - 1)
        def _():
            gmat = acc[...] / jnp.maximum(cnt[:, 0:1], 1.0)
            z = jnp.dot(gmat, l1w_ref[...],
                        preferred_element_type=jnp.float32,
                    precision=None) + l1b_ref[0:1, :]
            mu = jnp.mean(z, axis=0, keepdims=True)
            zc = z - mu
            var = jnp.mean(zc * zc, axis=0, keepdims=True)
            z = zc / jnp.sqrt(var + 1e-5) * bng_ref[0:1, :] + bnb_ref[0:1, :]
            z = jnp.maximum(z, 0.0)
            z2 = jnp.dot(z, l2w_ref[...],
                         preferred_element_type=jnp.float32,
                    precision=None) + l2b_ref[0:1, :]
            mx = jnp.max(z2, axis=1, keepdims=True)
            lse = jnp.log(jnp.sum(jnp.exp(z2 - mx), axis=1, keepdims=True)) + mx
            out_ref[...] = z2 - lse

    return pl.pallas_call(
        kern,
        grid=(NBLK,),
        in_specs=[
            pl.BlockSpec((BLK, HID2), lambda i: (i, 0)),
            pl.BlockSpec((8, HID), lambda i: (0, 0)),
            pl.BlockSpec((8, HID), lambda i: (0, 0)),
            pl.BlockSpec((8, HID), lambda i: (0, 0)),
            pl.BlockSpec((NC, BLK, HID2), lambda i: (0, i, 0)),
            pl.BlockSpec((1, 8, BLK), lambda i: (i, 0, 0)),
            pl.BlockSpec((HID, HID), lambda i: (0, 0)),
            pl.BlockSpec((8, HID), lambda i: (0, 0)),
            pl.BlockSpec((8, HID), lambda i: (0, 0)),
            pl.BlockSpec((8, HID), lambda i: (0, 0)),
            pl.BlockSpec((HID, 128), lambda i: (0, 0)),
            pl.BlockSpec((8, 128), lambda i: (0, 0)),
        ],
        out_specs=pl.BlockSpec((G, 128), lambda i: (0, 0)),
        out_shape=jax.ShapeDtypeStruct((G, 128), jnp.float32),
        scratch_shapes=[
            pltpu.VMEM((G, HID), jnp.float32),
            pltpu.VMEM((G, 128), jnp.float32),
        ],
    )(xlr2, cs2, att8, b8, acc2, batch3, l1w, l1b8, bng8, bnb8,
      l2wp, l2bp)


def _tile8(v):
    return jnp.tile(v.reshape(1, -1), (8, 1))


def kernel(x, edge_index, batch, edge_attr, Wl1, Wr1, We1, att1, b1,
           Wl2, Wr2, We2, att2, b2, lin1_W, lin1_b, bn_g, bn_b,
           lin2_W, lin2_b):
    src = jnp.concatenate(
        [edge_index[0], jnp.zeros((E_PAD - E,), jnp.int32)])
    dst = jnp.concatenate(
        [edge_index[1], jnp.zeros((E_PAD - E,), jnp.int32)])
    sd = jnp.concatenate(
        [src.reshape(E_PAD // K, K), dst.reshape(E_PAD // K, K)], axis=1)
    ea_pad = jnp.concatenate(
        [edge_attr, jnp.zeros((E_PAD - E, ED), jnp.float32)], axis=0)
    eat_pad = ea_pad.T
    zn = jnp.zeros((RPT, HID2), jnp.float32)

    xlr1 = _mm_lr(x, Wl1, Wr1)
    ewt1, cs1 = _edge_mm(ea_pad, eat_pad, We1, We1.T)
    acc1 = _get_sc_edge_pass()(sd, xlr1, ewt1, att1, zn)
    h = _self_and_norm(xlr1, cs1, _tile8(att1), _tile8(b1), acc1,
                       relu=True)

    xlr2 = _mm_lr(h, Wl2, Wr2)
    ewt2, cs2 = _edge_mm(ea_pad, eat_pad, We2, We2.T)
    acc2 = _get_sc_edge_pass()(sd, xlr2, ewt2, att2, zn)

    batch3 = jnp.tile(batch.astype(jnp.int32).reshape(NBLK, 1, BLK),
                      (1, 8, 1))
    l2wp = jnp.concatenate(
        [lin2_W, jnp.zeros((HID, 128 - OUT), jnp.float32)], axis=1)
    l2bp = jnp.concatenate(
        [lin2_b, jnp.full((128 - OUT,), -1e30, jnp.float32)])
    out = _head(xlr2, cs2, _tile8(att2), _tile8(b2), acc2, batch3,
                lin1_W, _tile8(lin1_b), _tile8(bn_g), _tile8(bn_b),
                l2wp, _tile8(l2bp))
    return out[:, :OUT]


# speedup vs baseline: 6.3558x; 1.3816x over previous
import functools

import jax
import jax.numpy as jnp
from jax import lax
from jax.experimental import pallas as pl
from jax.experimental.pallas import tpu as pltpu
from jax.experimental.pallas import tpu_sc as plsc

N = 10000
E = 320000
IN = 128
HID = 64
OUT = 10
ED = 16
G = 64

NC, NS, LANES = 2, 16, 16
NW = NC * NS
K = 64
NPAIR = -(-E // (NW * 2 * K))
NCHUNK = 2 * NPAIR
EPW = NCHUNK * K
E_PAD = EPW * NW
NPAD = 10112
RPT = NPAD // NS
DENW = 16
HID2 = 2 * HID
BLK = 1000
NBLK = N // BLK


def _bf16_round(x):
    bits = plsc.bitcast(x, jnp.int32)
    one = jnp.full(bits.shape, 1, jnp.int32)
    sixteen = jnp.full(bits.shape, 16, jnp.int32)
    lsb = lax.shift_right_logical(bits, sixteen) & one
    rounded = bits + 0x7FFF + lsb
    return plsc.bitcast(rounded & jnp.int32(-65536), jnp.float32)


def _exp_f32(x):
    log2e = 1.4426950408889634
    half = jnp.where(x >= 0, 0.5, -0.5)
    ni = (x * log2e + half).astype(jnp.int32)
    nf = ni.astype(jnp.float32)
    r = x - nf * 0.693359375 - nf * (-2.12194440e-4)
    p = 1.0 / 720.0
    p = p * r + 1.0 / 120.0
    p = p * r + 1.0 / 24.0
    p = p * r + 1.0 / 6.0
    p = p * r + 0.5
    p = p * r + 1.0
    p = p * r + 1.0
    scale = plsc.bitcast(
        lax.shift_left(ni + 127, jnp.full(ni.shape, 23, jnp.int32)),
        jnp.float32)
    return p * scale


@functools.cache
def _get_sc_edge_pass():
  mesh = plsc.VectorSubcoreMesh(core_axis_name="c", subcore_axis_name="s",
                                num_cores=NC, num_subcores=NS)

  @functools.partial(
    pl.kernel,
    out_type=jax.ShapeDtypeStruct((NC, NPAD, HID2), jnp.float32),
    mesh=mesh,
    scratch_types=[
        pltpu.VMEM((2 * K,), jnp.int32),
        pltpu.VMEM((2 * K,), jnp.int32),
        pltpu.VMEM((K,), jnp.int32),
        pltpu.VMEM((K,), jnp.int32),
        pltpu.VMEM((K, HID2), jnp.float32),
        pltpu.VMEM((K, HID2), jnp.float32),
        pltpu.VMEM((K, HID2), jnp.float32),
        pltpu.VMEM((K, HID2), jnp.float32),
        pltpu.VMEM((HID, 2 * K), jnp.float32),
        pltpu.VMEM((K,), jnp.float32),
        pltpu.VMEM((HID,), jnp.float32),
        pltpu.VMEM_SHARED((NPAD, HID2), jnp.float32),
        pltpu.SemaphoreType.DMA,
        pltpu.SemaphoreType.DMA,
        pltpu.SemaphoreType.DMA,
        pltpu.SemaphoreType.DMA,
    ],
    compiler_params=pltpu.CompilerParams(needs_layout_passes=False),
  )
  def _sc_edge_pass(sd_hbm, xlr_hbm, ewt_hbm, att_hbm, zn_hbm, acc_out,
                    sd_a, sd_b, dsti_a, dsti_b, g1_a, g2_a, g1_b, g2_b,
                    ewt_v, lbuf, att_v,
                    sh_acc, sa1, sa2, sb1, sb2):
      c = lax.axis_index("c")
      s = lax.axis_index("s")
      wid = c * NS + s
      row0 = s * RPT

      pltpu.sync_copy(zn_hbm, sh_acc.at[pl.ds(row0, RPT)])
      pltpu.sync_copy(att_hbm, att_v)
      plsc.subcore_barrier()

      iota16 = lax.iota(jnp.int32, LANES)
      col64 = jnp.full((LANES,), HID, jnp.int32)
      att_regs = [_bf16_round(att_v[pl.ds(q * LANES, LANES)])
                  for q in range(HID // LANES)]

      def prefetch(ci, sd_x, dsti_x, g1_x, g2_x, s1, s2):
          pltpu.sync_copy(sd_hbm.at[ci], sd_x)
          for q in range(K // LANES):
              dsti_x[pl.ds(q * LANES, LANES)] = (
                  sd_x[pl.ds(K + q * LANES, LANES)])
          pltpu.async_copy(xlr_hbm.at[sd_x.at[pl.ds(0, K)]], g1_x, s1)
          pltpu.async_copy(xlr_hbm.at[dsti_x], g2_x, s2)

      def compute(u, base, dsti_x, g1_x, g2_x):
          def lgroup(g, carry2):
              eids = g * LANES + iota16
              accs = [jnp.zeros((LANES,), jnp.float32) for _ in range(4)]
              for h in range(HID):
                  hv = jnp.full((LANES,), h, jnp.int32)
                  hv2 = jnp.full((LANES,), HID + h, jnp.int32)
                  m = (plsc.load_gather(g1_x, [eids, hv])
                       + plsc.load_gather(g2_x, [eids, hv2])
                       + ewt_v[h, pl.ds(u * K + g * LANES, LANES)])
                  m = jnp.maximum(m, 0.2 * m)
                  m = _bf16_round(m)
                  accs[h % 4] = accs[h % 4] + m * att_regs[h // LANES][h % LANES]
              acc = (accs[0] + accs[1]) + (accs[2] + accs[3])
              gi = base + g * LANES + iota16
              ex = jnp.where(gi < E, _exp_f32(acc), 0.0)
              lbuf[pl.ds(g * LANES, LANES)] = ex
              plsc.store_scatter(g2_x, [eids, col64], ex)
              return carry2

          lax.fori_loop(0, K // LANES, lgroup, 0)

          def wgroup(g, carry2):
              exv = lbuf[pl.ds(g * LANES, LANES)]
              for uu in range(LANES):
                  e = g * LANES + uu
                  w = exv[uu]
                  for q in range(HID // LANES):
                      sl = pl.ds(q * LANES, LANES)
                      g2_x[e, sl] = g1_x[e, sl] * w
              return carry2

          lax.fori_loop(0, K // LANES, wgroup, 0)
          pltpu.sync_copy(g2_x, sh_acc.at[dsti_x], add=True)

      prefetch(wid * NCHUNK, sd_a, dsti_a, g1_a, g2_a, sa1, sa2)

      def pair(t, carry):
          ci0 = (wid * NPAIR + t) * 2
          pbase = ci0 * K
          pltpu.sync_copy(ewt_hbm.at[:, pl.ds(pbase, 2 * K)], ewt_v)
          prefetch(ci0 + 1, sd_b, dsti_b, g1_b, g2_b, sb1, sb2)

          pltpu.make_async_copy(xlr_hbm.at[sd_a.at[pl.ds(0, K)]],
                                g1_a, sa1).wait()
          pltpu.make_async_copy(xlr_hbm.at[dsti_a], g2_a, sa2).wait()
          compute(0, ci0 * K, dsti_a, g1_a, g2_a)

          @pl.when(t < NPAIR - 1)
          def _():
              prefetch(ci0 + 2, sd_a, dsti_a, g1_a, g2_a, sa1, sa2)

          pltpu.make_async_copy(xlr_hbm.at[sd_b.at[pl.ds(0, K)]],
                                g1_b, sb1).wait()
          pltpu.make_async_copy(xlr_hbm.at[dsti_b], g2_b, sb2).wait()
          compute(1, (ci0 + 1) * K, dsti_b, g1_b, g2_b)
          return carry

      lax.fori_loop(0, NPAIR, pair, 0)
      plsc.subcore_barrier()

      sl = pl.ds(row0, RPT)
      pltpu.sync_copy(sh_acc.at[sl], acc_out.at[c, sl])

  return _sc_edge_pass


def _mm_lr(x, wl, wr):
    n, d_in = x.shape

    def kern(x_ref, wl_ref, wr_ref, o_ref):
        xv = x_ref[...]
        o_ref[:, 0:HID] = jnp.dot(xv, wl_ref[...],
                                  preferred_element_type=jnp.float32,
                    precision=None)
        o_ref[:, HID:HID2] = jnp.dot(xv, wr_ref[...],
                                     preferred_element_type=jnp.float32,
                    precision=None)

    return pl.pallas_call(
        kern,
        grid=(n // BLK,),
        in_specs=[
            pl.BlockSpec((BLK, d_in), lambda i: (i, 0)),
            pl.BlockSpec((d_in, HID), lambda i: (0, 0)),
            pl.BlockSpec((d_in, HID), lambda i: (0, 0)),
        ],
        out_specs=pl.BlockSpec((BLK, HID2), lambda i: (i, 0)),
        out_shape=jax.ShapeDtypeStruct((n, HID2), jnp.float32),
    )(x, wl, wr)


def _edge_mm(ea_pad, eat_pad, we, wet):
    eblk = 4096
    ngrid = E_PAD // eblk

    def kern(ea_ref, eat_ref, we_ref, wet_ref, ewt_ref, cs_ref, csea):
        i = pl.program_id(0)
        ewt_ref[...] = jnp.dot(wet_ref[...], eat_ref[...],
                               preferred_element_type=jnp.float32,
                    precision=None)

        @pl.when(i == 0)
        def _():
            csea[...] = jnp.zeros_like(csea)

        csea[...] += jnp.sum(ea_ref[...], axis=0, keepdims=True)

        @pl.when(i == ngrid - 1)
        def _():
            cs_ref[...] = jnp.dot(csea[...], we_ref[...],
                                  preferred_element_type=jnp.float32,
                    precision=None)

    return pl.pallas_call(
        kern,
        grid=(ngrid,),
        in_specs=[
            pl.BlockSpec((eblk, ED), lambda i: (i, 0)),
            pl.BlockSpec((ED, eblk), lambda i: (0, i)),
            pl.BlockSpec((ED, HID), lambda i: (0, 0)),
            pl.BlockSpec((HID, ED), lambda i: (0, 0)),
        ],
        out_specs=[
            pl.BlockSpec((HID, eblk), lambda i: (0, i)),
            pl.BlockSpec((8, HID), lambda i: (0, 0)),
        ],
        out_shape=[
            jax.ShapeDtypeStruct((HID, E_PAD), jnp.float32),
            jax.ShapeDtypeStruct((8, HID), jnp.float32),
        ],
        scratch_shapes=[pltpu.VMEM((8, ED), jnp.float32)],
    )(ea_pad, eat_pad, we, wet)


def _self_and_norm(xlr, cs, att8, b8, acc_, relu):
    def kern(xlr_ref, cs_ref, att_ref, b_ref, acc_ref, h_ref):
        xlrv = xlr_ref[...]
        xlv = xlrv[:, 0:HID]
        xrv = xlrv[:, HID:HID2]
        mean_w = cs_ref[0:1, :] * (1.0 / E)
        ms = xlv + xrv + mean_w
        ms = jnp.maximum(ms, 0.2 * ms)
        ms = ms.astype(jnp.bfloat16).astype(jnp.float32)
        attv = att_ref[0:1, :].astype(jnp.bfloat16).astype(jnp.float32)
        sv = jnp.exp(jnp.sum(ms * attv, axis=1, keepdims=True))
        num = sv * xlv + acc_ref[0, :, 0:HID] + acc_ref[1, :, 0:HID]
        den = sv + acc_ref[0, :, HID:HID + 1] + acc_ref[1, :, HID:HID + 1]
        h = num / den + b_ref[0:1, :]
        if relu:
            h = jnp.maximum(h, 0.0)
        h_ref[...] = h

    return pl.pallas_call(
        functools.partial(kern),
        grid=(NBLK,),
        in_specs=[
            pl.BlockSpec((BLK, HID2), lambda i: (i, 0)),
            pl.BlockSpec((8, HID), lambda i: (0, 0)),
            pl.BlockSpec((8, HID), lambda i: (0, 0)),
            pl.BlockSpec((8, HID), lambda i: (0, 0)),
            pl.BlockSpec((NC, BLK, HID2), lambda i: (0, i, 0)),
        ],
        out_specs=pl.BlockSpec((BLK, HID), lambda i: (i, 0)),
        out_shape=jax.ShapeDtypeStruct((N, HID), jnp.float32),
    )(xlr, cs, att8, b8, acc_)


def _head(xlr2, cs2, att8, b8, acc2, batch3, l1w, l1b8, bng8, bnb8,
          l2wp, l2bp):
    def kern(xlr_ref, cs_ref, att_ref, b_ref, acc_ref, bat_ref,
             l1w_ref, l1b_ref, bng_ref, bnb_ref, l2w_ref, l2b_ref, out_ref,
             acc, cnt):
        i = pl.program_id(0)

        @pl.when(i == 0)
        def _():
            acc[...] = jnp.zeros_like(acc)
            cnt[...] = jnp.zeros_like(cnt)

        xlrv = xlr_ref[...]
        xlv = xlrv[:, 0:HID]
        xrv = xlrv[:, HID:HID2]
        mean_w = cs_ref[0:1, :] * (1.0 / E)
        ms = xlv + xrv + mean_w
        ms = jnp.maximum(ms, 0.2 * ms)
        ms = ms.astype(jnp.bfloat16).astype(jnp.float32)
        attv = att_ref[0:1, :].astype(jnp.bfloat16).astype(jnp.float32)
        sv = jnp.exp(jnp.sum(ms * attv, axis=1, keepdims=True))
        num = sv * xlv + acc_ref[0, :, 0:HID] + acc_ref[1, :, 0:HID]
        den = sv + acc_ref[0, :, HID:HID + 1] + acc_ref[1, :, HID:HID + 1]
        h = num / den + b_ref[0:1, :]

        one_hot = (bat_ref[0, 0:1, :] ==
                   lax.broadcasted_iota(jnp.int32, (G, BLK), 0)
                   ).astype(jnp.float32)
        acc[...] += jnp.dot(one_hot, h, preferred_element_type=jnp.float32,
                    precision=lax.Precision.HIGHEST)
        cnt[:, 0:1] += jnp.sum(one_hot, axis=1, keepdims=True)

        @pl.when(i == NBLK - 1)
        def _():
            gmat = acc[...] / jnp.maximum(cnt[:, 0:1], 1.0)
            z = jnp.dot(gmat, l1w_ref[...],
                        preferred_element_type=jnp.float32,
                    precision=None) + l1b_ref[0:1, :]
            mu = jnp.mean(z, axis=0, keepdims=True)
            zc = z - mu
            var = jnp.mean(zc * zc, axis=0, keepdims=True)
            z = zc / jnp.sqrt(var + 1e-5) * bng_ref[0:1, :] + bnb_ref[0:1, :]
            z = jnp.maximum(z, 0.0)
            z2 = jnp.dot(z, l2w_ref[...],
                         preferred_element_type=jnp.float32,
                    precision=None) + l2b_ref[0:1, :]
            mx = jnp.max(z2, axis=1, keepdims=True)
            lse = jnp.log(jnp.sum(jnp.exp(z2 - mx), axis=1, keepdims=True)) + mx
            out_ref[...] = z2 - lse

    return pl.pallas_call(
        kern,
        grid=(NBLK,),
        in_specs=[
            pl.BlockSpec((BLK, HID2), lambda i: (i, 0)),
            pl.BlockSpec((8, HID), lambda i: (0, 0)),
            pl.BlockSpec((8, HID), lambda i: (0, 0)),
            pl.BlockSpec((8, HID), lambda i: (0, 0)),
            pl.BlockSpec((NC, BLK, HID2), lambda i: (0, i, 0)),
            pl.BlockSpec((1, 8, BLK), lambda i: (i, 0, 0)),
            pl.BlockSpec((HID, HID), lambda i: (0, 0)),
            pl.BlockSpec((8, HID), lambda i: (0, 0)),
            pl.BlockSpec((8, HID), lambda i: (0, 0)),
            pl.BlockSpec((8, HID), lambda i: (0, 0)),
            pl.BlockSpec((HID, 128), lambda i: (0, 0)),
            pl.BlockSpec((8, 128), lambda i: (0, 0)),
        ],
        out_specs=pl.BlockSpec((G, 128), lambda i: (0, 0)),
        out_shape=jax.ShapeDtypeStruct((G, 128), jnp.float32),
        scratch_shapes=[
            pltpu.VMEM((G, HID), jnp.float32),
            pltpu.VMEM((G, 128), jnp.float32),
        ],
    )(xlr2, cs2, att8, b8, acc2, batch3, l1w, l1b8, bng8, bnb8,
      l2wp, l2bp)


def _tile8(v):
    return jnp.tile(v.reshape(1, -1), (8, 1))


def kernel(x, edge_index, batch, edge_attr, Wl1, Wr1, We1, att1, b1,
           Wl2, Wr2, We2, att2, b2, lin1_W, lin1_b, bn_g, bn_b,
           lin2_W, lin2_b):
    src = jnp.concatenate(
        [edge_index[0], jnp.zeros((E_PAD - E,), jnp.int32)])
    dst = jnp.concatenate(
        [edge_index[1], jnp.zeros((E_PAD - E,), jnp.int32)])
    sd = jnp.concatenate(
        [src.reshape(E_PAD // K, K), dst.reshape(E_PAD // K, K)], axis=1)
    ea_pad = jnp.concatenate(
        [edge_attr, jnp.zeros((E_PAD - E, ED), jnp.float32)], axis=0)
    eat_pad = ea_pad.T
    zn = jnp.zeros((RPT, HID2), jnp.float32)

    xlr1 = _mm_lr(x, Wl1, Wr1)
    ewt1, cs1 = _edge_mm(ea_pad, eat_pad, We1, We1.T)
    acc1 = _get_sc_edge_pass()(sd, xlr1, ewt1, att1, zn)
    h = _self_and_norm(xlr1, cs1, _tile8(att1), _tile8(b1), acc1,
                       relu=True)

    xlr2 = _mm_lr(h, Wl2, Wr2)
    ewt2, cs2 = _edge_mm(ea_pad, eat_pad, We2, We2.T)
    acc2 = _get_sc_edge_pass()(sd, xlr2, ewt2, att2, zn)

    batch3 = jnp.tile(batch.astype(jnp.int32).reshape(NBLK, 1, BLK),
                      (1, 8, 1))
    l2wp = jnp.concatenate(
        [lin2_W, jnp.zeros((HID, 128 - OUT), jnp.float32)], axis=1)
    l2bp = jnp.concatenate(
        [lin2_b, jnp.full((128 - OUT,), -1e30, jnp.float32)])
    out = _head(xlr2, cs2, _tile8(att2), _tile8(b2), acc2, batch3,
                lin1_W, _tile8(lin1_b), _tile8(bn_g), _tile8(bn_b),
                l2wp, _tile8(l2bp))
    return out[:, :OUT]
